# baseline (speedup 1.0000x reference)
import jax
import jax.numpy as jnp
from jax import lax
from jax.experimental import pallas as pl
from jax.experimental.pallas import tpu as pltpu
from jax.experimental.pallas import tpu_sc as plsc

_REG = 0.001
_B = 16384
_D = 32
_NW = 32
_BPW = _B // _NW
_CHUNK = 128
_NCHUNK = _BPW // _CHUNK
_GROUPS = _BPW // 16


def _sc_body(rows_hbm, cols_hbm, rat_hbm, sen_hbm, w_hbm, p_hbm, q_hbm,
             out_hbm, ridx_refs, cidx_refs, ues_v, uis_v, rat_v, sen_v, w_v,
             outv, sem):
    wid = lax.axis_index("s") * 2 + lax.axis_index("c")
    base = wid * _BPW

    for j in range(_NCHUNK):
        pltpu.sync_copy(rows_hbm.at[pl.ds(base + j * _CHUNK, _CHUNK)], ridx_refs[j])
        pltpu.sync_copy(cols_hbm.at[pl.ds(base + j * _CHUNK, _CHUNK)], cidx_refs[j])

    copies = []
    for j in range(_NCHUNK):
        sl = pl.ds(j * _CHUNK, _CHUNK)
        copies.append(pltpu.async_copy(p_hbm.at[ridx_refs[j]], ues_v.at[sl], sem))
        copies.append(pltpu.async_copy(q_hbm.at[cidx_refs[j]], uis_v.at[sl], sem))

    pltpu.sync_copy(rat_hbm.at[pl.ds(base, _BPW)], rat_v)
    pltpu.sync_copy(sen_hbm.at[pl.ds(base, _BPW)], sen_v)
    pltpu.sync_copy(w_hbm.at[pl.ds(base, _BPW)], w_v)

    for cp in copies:
        cp.wait()

    lane = lax.iota(jnp.int32, 16)
    zero = jnp.zeros((16,), jnp.float32)
    masks = [lane == j for j in range(16)]
    perms = [lane ^ sh for sh in (8, 4, 2, 1)]

    def group_body(g, carry):
        lossacc, regacc = carry
        dots = zero
        for j in range(16):
            r = g * 16 + j
            a0 = ues_v[r, pl.ds(0, 16)]
            a1 = ues_v[r, pl.ds(16, 16)]
            b0 = uis_v[r, pl.ds(0, 16)]
            b1 = uis_v[r, pl.ds(16, 16)]
            s = a0 * b0 + a1 * b1
            for p in perms:
                s = s + s.at[p].get(mode="promise_in_bounds")
            dots = jnp.where(masks[j], s, dots)
            regacc = regacc + (a0 * a0 + a1 * a1 + b0 * b0 + b1 * b1)
        off = g * 16
        rat = rat_v[pl.ds(off, 16)]
        sen = sen_v[pl.ds(off, 16)]
        w2 = w_v[pl.ds(off, 16)] - 0.0001
        pr = 1.0 / (1.0 + jnp.exp(-dots))
        e1 = pr - rat
        e2 = pr - sen
        lossacc = lossacc + e1 * e1 * w2 + e2 * e2 * (1.0 - w2)
        return lossacc, regacc

    lossacc, regacc = lax.fori_loop(0, _GROUPS, group_body, (zero, zero))
    outv[...] = lossacc + _REG * regacc
    pltpu.sync_copy(outv, out_hbm.at[wid])


@jax.jit
def kernel(rows, cols, ratval, senval, wval, P, Q):
    mesh = plsc.VectorSubcoreMesh(
        core_axis_name="c", subcore_axis_name="s", num_cores=2, num_subcores=16
    )
    partials = pl.kernel(
        _sc_body,
        out_type=jax.ShapeDtypeStruct((_NW, 16), jnp.float32),
        mesh=mesh,
        compiler_params=pltpu.CompilerParams(use_tc_tiling_on_sc=False),
        scratch_types=[
            [pltpu.VMEM((_CHUNK,), jnp.int32) for _ in range(_NCHUNK)],
            [pltpu.VMEM((_CHUNK,), jnp.int32) for _ in range(_NCHUNK)],
            pltpu.VMEM((_BPW, _D), jnp.float32),
            pltpu.VMEM((_BPW, _D), jnp.float32),
            pltpu.VMEM((_BPW,), jnp.float32),
            pltpu.VMEM((_BPW,), jnp.float32),
            pltpu.VMEM((_BPW,), jnp.float32),
            pltpu.VMEM((16,), jnp.float32),
            pltpu.SemaphoreType.DMA,
        ],
    )(rows, cols, ratval, senval, wval, P, Q)
    return jnp.sum(partials)
